# baseline (device time: 101833 ns/iter reference)
import jax
import jax.numpy as jnp
from jax import lax
from jax.experimental import pallas as pl
from jax.experimental.pallas import tpu as pltpu

N_DEV = 4


def kernel(x, W1, W2):
    m, k = x.shape
    h = W1.shape[1]
    n = W2.shape[1]
    chunk = m // N_DEV

    def body(x_ref, w1_ref, w2_ref, out_ref,
             send_buf, rs_recv, ag_buf,
             rs_send_sems, rs_recv_sems, ag_send_sems, ag_recv_sems):
        me = lax.axis_index("i")
        left = (me - 1) % N_DEV
        right = (me + 1) % N_DEV

        barrier_sem = pltpu.get_barrier_semaphore()
        for nbr in (left, right):
            pl.semaphore_signal(
                barrier_sem, inc=1,
                device_id=(nbr,), device_id_type=pltpu.DeviceIdType.MESH,
            )
        pl.semaphore_wait(barrier_sem, 2)

        def compute_chunk(c):
            xb = x_ref[pl.ds(c * chunk, chunk), :]
            hb = jnp.maximum(
                jnp.dot(xb, w1_ref[...], preferred_element_type=jnp.float32),
                0.0,
            )
            return jnp.dot(hb, w2_ref[...], preferred_element_type=jnp.float32)

        for s in range(N_DEV - 1):
            c_send = (me - s) % N_DEV
            p = compute_chunk(c_send)
            if s == 0:
                send_buf[...] = p
            else:
                send_buf[...] = p + rs_recv[s - 1]
            rdma = pltpu.make_async_remote_copy(
                src_ref=send_buf,
                dst_ref=rs_recv.at[s],
                send_sem=rs_send_sems.at[s],
                recv_sem=rs_recv_sems.at[s],
                device_id=(right,),
                device_id_type=pltpu.DeviceIdType.MESH,
            )
            rdma.start()
            rdma.wait()

        own_c = (me + 1) % N_DEV
        own = compute_chunk(own_c) + rs_recv[N_DEV - 2]
        send_buf[...] = own
        out_ref[pl.ds(own_c * chunk, chunk), :] = own

        for t in range(N_DEV - 1):
            src = send_buf if t == 0 else ag_buf.at[t - 1]
            rdma = pltpu.make_async_remote_copy(
                src_ref=src,
                dst_ref=ag_buf.at[t],
                send_sem=ag_send_sems.at[t],
                recv_sem=ag_recv_sems.at[t],
                device_id=(right,),
                device_id_type=pltpu.DeviceIdType.MESH,
            )
            rdma.start()
            rdma.wait()
            orig = (me - t) % N_DEV
            out_ref[pl.ds(orig * chunk, chunk), :] = ag_buf[t]

    return pl.pallas_call(
        body,
        out_shape=jax.ShapeDtypeStruct((m, n), jnp.float32),
        in_specs=[
            pl.BlockSpec(memory_space=pltpu.VMEM),
            pl.BlockSpec(memory_space=pltpu.VMEM),
            pl.BlockSpec(memory_space=pltpu.VMEM),
        ],
        out_specs=pl.BlockSpec(memory_space=pltpu.VMEM),
        scratch_shapes=[
            pltpu.VMEM((chunk, n), jnp.float32),
            pltpu.VMEM((N_DEV - 1, chunk, n), jnp.float32),
            pltpu.VMEM((N_DEV - 1, chunk, n), jnp.float32),
            pltpu.SemaphoreType.DMA((N_DEV - 1,)),
            pltpu.SemaphoreType.DMA((N_DEV - 1,)),
            pltpu.SemaphoreType.DMA((N_DEV - 1,)),
            pltpu.SemaphoreType.DMA((N_DEV - 1,)),
        ],
        compiler_params=pltpu.CompilerParams(collective_id=0),
    )(x, W1, W2)


# device time: 60757 ns/iter; 1.6761x vs baseline; 1.6761x over previous
import jax
import jax.numpy as jnp
from jax import lax
from jax.experimental import pallas as pl
from jax.experimental.pallas import tpu as pltpu

N_DEV = 4


def kernel(x, W1, W2):
    m, k = x.shape
    n = W2.shape[1]
    chunk = m // N_DEV
    half = n // 2

    def body(x_ref, w1_ref, w2_ref, out_ref,
             pc, sb, rs, ag, own_buf,
             rs_send_sems, rs_recv_sems, ag_send_sems, ag_recv_sems):
        me = lax.axis_index("i")
        left = (me - 1) % N_DEV
        right = (me + 1) % N_DEV

        barrier_sem = pltpu.get_barrier_semaphore()
        for nbr in (left, right):
            pl.semaphore_signal(
                barrier_sem, inc=1,
                device_id=(nbr,), device_id_type=pltpu.DeviceIdType.MESH,
            )
        pl.semaphore_wait(barrier_sem, 2)

        def compute_chunk(c):
            xb = x_ref[pl.ds(c * chunk, chunk), :]
            hb = jnp.maximum(
                jnp.dot(xb, w1_ref[...], preferred_element_type=jnp.float32),
                0.0,
            )
            return jnp.dot(hb, w2_ref[...], preferred_element_type=jnp.float32)

        def rs_rdma(d, s):
            tgt = right if d == 0 else left
            return pltpu.make_async_remote_copy(
                src_ref=sb.at[d, s],
                dst_ref=rs.at[d, s],
                send_sem=rs_send_sems.at[d, s],
                recv_sem=rs_recv_sems.at[d, s],
                device_id=(tgt,),
                device_id_type=pltpu.DeviceIdType.MESH,
            )

        p = compute_chunk(me)
        sb[0, 0] = p[:, half:]
        sb[1, 0] = p[:, :half]
        r0 = rs_rdma(0, 0)
        l0 = rs_rdma(1, 0)
        r0.start()
        l0.start()

        pc[0] = compute_chunk((me - 1) % N_DEV)
        pc[1] = compute_chunk((me + 1) % N_DEV)

        r0.wait_recv()
        l0.wait_recv()
        sb[0, 1] = pc[0][:, half:] + rs[0, 0]
        sb[1, 1] = pc[1][:, :half] + rs[1, 0]
        r1 = rs_rdma(0, 1)
        l1 = rs_rdma(1, 1)
        r1.start()
        l1.start()

        pc[2] = compute_chunk((me + 2) % N_DEV)

        r1.wait_recv()
        l1.wait_recv()
        sb[0, 2] = pc[2][:, half:] + rs[0, 1]
        sb[1, 2] = pc[2][:, :half] + rs[1, 1]
        r2 = rs_rdma(0, 2)
        l2 = rs_rdma(1, 2)
        r2.start()
        l2.start()

        r2.wait_recv()
        l2.wait_recv()
        own_r_c = (me + 1) % N_DEV
        own_l_c = (me - 1) % N_DEV
        own_buf[0] = pc[1][:, half:] + rs[0, 2]
        own_buf[1] = pc[0][:, :half] + rs[1, 2]
        out_ref[pl.ds(own_r_c * chunk, chunk), pl.ds(half, half)] = own_buf[0]
        out_ref[pl.ds(own_l_c * chunk, chunk), pl.ds(0, half)] = own_buf[1]

        for rd in (r0, l0, r1, l1, r2, l2):
            rd.wait_send()

        ag_rdmas = []
        for t in range(N_DEV - 1):
            src_r = own_buf.at[0] if t == 0 else ag.at[0, t - 1]
            src_l = own_buf.at[1] if t == 0 else ag.at[1, t - 1]
            agr = pltpu.make_async_remote_copy(
                src_ref=src_r, dst_ref=ag.at[0, t],
                send_sem=ag_send_sems.at[0, t], recv_sem=ag_recv_sems.at[0, t],
                device_id=(right,), device_id_type=pltpu.DeviceIdType.MESH,
            )
            agl = pltpu.make_async_remote_copy(
                src_ref=src_l, dst_ref=ag.at[1, t],
                send_sem=ag_send_sems.at[1, t], recv_sem=ag_recv_sems.at[1, t],
                device_id=(left,), device_id_type=pltpu.DeviceIdType.MESH,
            )
            agr.start()
            agl.start()
            ag_rdmas += [agr, agl]
            agr.wait_recv()
            agl.wait_recv()
            orig_r = (me - t) % N_DEV
            orig_l = (me + t) % N_DEV
            out_ref[pl.ds(orig_r * chunk, chunk), pl.ds(half, half)] = ag[0, t]
            out_ref[pl.ds(orig_l * chunk, chunk), pl.ds(0, half)] = ag[1, t]

        for rd in ag_rdmas:
            rd.wait_send()

    return pl.pallas_call(
        body,
        out_shape=jax.ShapeDtypeStruct((m, n), jnp.float32),
        in_specs=[
            pl.BlockSpec(memory_space=pltpu.VMEM),
            pl.BlockSpec(memory_space=pltpu.VMEM),
            pl.BlockSpec(memory_space=pltpu.VMEM),
        ],
        out_specs=pl.BlockSpec(memory_space=pltpu.VMEM),
        scratch_shapes=[
            pltpu.VMEM((3, chunk, n), jnp.float32),
            pltpu.VMEM((2, 3, chunk, half), jnp.float32),
            pltpu.VMEM((2, 3, chunk, half), jnp.float32),
            pltpu.VMEM((2, 3, chunk, half), jnp.float32),
            pltpu.VMEM((2, chunk, half), jnp.float32),
            pltpu.SemaphoreType.DMA((2, 3)),
            pltpu.SemaphoreType.DMA((2, 3)),
            pltpu.SemaphoreType.DMA((2, 3)),
            pltpu.SemaphoreType.DMA((2, 3)),
        ],
        compiler_params=pltpu.CompilerParams(collective_id=0),
    )(x, W1, W2)


# device time: 51493 ns/iter; 1.9776x vs baseline; 1.1799x over previous
import jax
import jax.numpy as jnp
from jax import lax
from jax.experimental import pallas as pl
from jax.experimental.pallas import tpu as pltpu

N_DEV = 4
K = 2
R, L = 0, 1


def kernel(x, W1, W2):
    m, k = x.shape
    n = W2.shape[1]
    chunk = m // N_DEV
    sub = chunk // K
    half = n // 2

    def body(x_ref, w1_ref, w2_ref, out_ref,
             pc, sb, rs, ag, own,
             rs_ssem, rs_rsem, ag_ssem, ag_rsem):
        me = lax.axis_index("i")
        left = (me - 1) % N_DEV
        right = (me + 1) % N_DEV

        barrier_sem = pltpu.get_barrier_semaphore()
        for nbr in (left, right):
            pl.semaphore_signal(
                barrier_sem, inc=1,
                device_id=(nbr,), device_id_type=pltpu.DeviceIdType.MESH,
            )
        pl.semaphore_wait(barrier_sem, 2)

        def compute_sub(c, j):
            xb = x_ref[pl.ds(c * chunk + j * sub, sub), :]
            hb = jnp.maximum(
                jnp.dot(xb, w1_ref[...], preferred_element_type=jnp.float32),
                0.0,
            )
            return jnp.dot(hb, w2_ref[...], preferred_element_type=jnp.float32)

        def tgt(d):
            return right if d == R else left

        def hcols(v, d):
            return v[:, half:] if d == R else v[:, :half]

        started = []
        rs_desc = {}
        ag_desc = {}

        def start_rs(d, s, j, val):
            sb[d, s, j] = val
            r = pltpu.make_async_remote_copy(
                src_ref=sb.at[d, s, j], dst_ref=rs.at[d, s, j],
                send_sem=rs_ssem.at[d, s, j], recv_sem=rs_rsem.at[d, s, j],
                device_id=(tgt(d),), device_id_type=pltpu.DeviceIdType.MESH,
            )
            r.start()
            rs_desc[(d, s, j)] = r
            started.append(r)

        def start_ag(d, t, j):
            src = own.at[d, j] if t == 0 else ag.at[d, t - 1, j]
            a = pltpu.make_async_remote_copy(
                src_ref=src, dst_ref=ag.at[d, t, j],
                send_sem=ag_ssem.at[d, t, j], recv_sem=ag_rsem.at[d, t, j],
                device_id=(tgt(d),), device_id_type=pltpu.DeviceIdType.MESH,
            )
            a.start()
            ag_desc[(d, t, j)] = a
            started.append(a)

        cidx = [(me - 1) % N_DEV, (me + 1) % N_DEV, (me + 2) % N_DEV]
        rows = [slice(j * sub, (j + 1) * sub) for j in range(K)]

        for j in range(K):
            p = compute_sub(me, j)
            start_rs(R, 0, j, hcols(p, R))
            start_rs(L, 0, j, hcols(p, L))

        for j in range(K):
            pc[0, rows[j]] = compute_sub(cidx[0], j)
            rs_desc[(R, 0, j)].wait_recv()
            start_rs(R, 1, j, pc[0, rows[j], half:] + rs[R, 0, j])
            pc[1, rows[j]] = compute_sub(cidx[1], j)
            rs_desc[(L, 0, j)].wait_recv()
            start_rs(L, 1, j, pc[1, rows[j], :half] + rs[L, 0, j])

        for j in range(K):
            pc[2, rows[j]] = compute_sub(cidx[2], j)
            rs_desc[(R, 1, j)].wait_recv()
            start_rs(R, 2, j, pc[2, rows[j], half:] + rs[R, 1, j])
            rs_desc[(L, 1, j)].wait_recv()
            start_rs(L, 2, j, pc[2, rows[j], :half] + rs[L, 1, j])

        own_c = {R: (me + 1) % N_DEV, L: (me - 1) % N_DEV}
        own_pc = {R: 1, L: 0}
        for j in range(K):
            for d in (R, L):
                rs_desc[(d, 2, j)].wait_recv()
                own[d, j] = hcols(pc[own_pc[d], rows[j]], d) + rs[d, 2, j]
                col0 = half if d == R else 0
                out_ref[pl.ds(own_c[d] * chunk + j * sub, sub),
                        pl.ds(col0, half)] = own[d, j]
                start_ag(d, 0, j)

        def ag_orig(d, t):
            return (me - t) % N_DEV if d == R else (me + t) % N_DEV

        for t in (1, 2):
            for j in range(K):
                for d in (R, L):
                    ag_desc[(d, t - 1, j)].wait_recv()
                    col0 = half if d == R else 0
                    out_ref[pl.ds(ag_orig(d, t - 1) * chunk + j * sub, sub),
                            pl.ds(col0, half)] = ag[d, t - 1, j]
                    start_ag(d, t, j)

        for j in range(K):
            for d in (R, L):
                ag_desc[(d, 2, j)].wait_recv()
                col0 = half if d == R else 0
                out_ref[pl.ds(ag_orig(d, 2) * chunk + j * sub, sub),
                        pl.ds(col0, half)] = ag[d, 2, j]

        for r in started:
            r.wait_send()

    return pl.pallas_call(
        body,
        out_shape=jax.ShapeDtypeStruct((m, n), jnp.float32),
        in_specs=[
            pl.BlockSpec(memory_space=pltpu.VMEM),
            pl.BlockSpec(memory_space=pltpu.VMEM),
            pl.BlockSpec(memory_space=pltpu.VMEM),
        ],
        out_specs=pl.BlockSpec(memory_space=pltpu.VMEM),
        scratch_shapes=[
            pltpu.VMEM((3, chunk, n), jnp.float32),
            pltpu.VMEM((2, 3, K, sub, half), jnp.float32),
            pltpu.VMEM((2, 3, K, sub, half), jnp.float32),
            pltpu.VMEM((2, 3, K, sub, half), jnp.float32),
            pltpu.VMEM((2, K, sub, half), jnp.float32),
            pltpu.SemaphoreType.DMA((2, 3, K)),
            pltpu.SemaphoreType.DMA((2, 3, K)),
            pltpu.SemaphoreType.DMA((2, 3, K)),
            pltpu.SemaphoreType.DMA((2, 3, K)),
        ],
        compiler_params=pltpu.CompilerParams(collective_id=0),
    )(x, W1, W2)
